# Initial kernel scaffold; baseline (speedup 1.0000x reference)
import jax
import jax.numpy as jnp
from jax.experimental import pallas as pl


def kernel(features, labels):
    raise NotImplementedError("write your pallas kernel here")


# speedup vs baseline: 3.0632x; 3.0632x over previous
import functools

import jax
import jax.numpy as jnp
from jax import lax
from jax.experimental import pallas as pl
from jax.experimental.pallas import tpu as pltpu
from jax.experimental.pallas import tpu_sc as plsc

K = 50
GAMMA = 1.0
ALPHA = 0.99
MAX_ITER = 20
KPAD = 64
RB1 = 128
RB2 = 256
NEG = -jnp.inf


def _topk_body(xb_ref, xf_ref, i_ref, d_ref, rs_ref, sim_s, vals_s, idxs_s, *, n):
    b = pl.program_id(0)
    sim_s[...] = jax.lax.dot_general(
        xb_ref[...], xf_ref[...], (((1,), (1,)), ((), ())),
        preferred_element_type=jnp.float32)
    col = jax.lax.broadcasted_iota(jnp.int32, (RB1, n), 1)

    def ext(k, _):
        s = sim_s[...]
        m = jnp.max(s, axis=1)
        idx = jnp.argmax(s, axis=1).astype(jnp.int32)
        vals_s[pl.ds(k, 1), :] = m[None, :]
        idxs_s[pl.ds(k, 1), :] = idx[None, :]
        sim_s[...] = jnp.where(col == idx[:, None], NEG, s)
        return 0

    jax.lax.fori_loop(0, K + 1, ext, 0)

    rowids = b * RB1 + jax.lax.broadcasted_iota(jnp.int32, (RB1, 1), 0)
    valsT = vals_s[...].T
    idxsT = idxs_s[...].T
    dmax = valsT[:, 1:2]
    dmin = valsT[:, K:K + 1]
    inv = 1.0 / (dmax - dmin + 1e-8)
    dn = (valsT[:, 1:K + 1] - dmin) * inv
    iv = idxsT[:, 1:K + 1]
    keep = iv != rowids
    iv = jnp.where(keep, iv, rowids)
    dn = jnp.where(keep, dn, 0.0)
    padi = jnp.broadcast_to(rowids, (RB1, KPAD - K))
    padd = jnp.zeros((RB1, KPAD - K), jnp.float32)
    i_ref[...] = jnp.concatenate([iv, padi], axis=1)
    d_ref[...] = jnp.concatenate([dn, padd], axis=1)
    rs_ref[...] = jnp.sum(dn, axis=1, keepdims=True)


NW = 32
GB = 8


def _sc_build_body(i_hbm, d_hbm, a_hbm, cs_hbm, ivm, dvm, rowbuf, colacc, *, n):
    rpb = n // NW
    wid = lax.axis_index("s") * 2 + lax.axis_index("c")
    base = wid * rpb
    pltpu.sync_copy(i_hbm.at[pl.ds(base * KPAD, rpb * KPAD)], ivm)
    pltpu.sync_copy(d_hbm.at[pl.ds(base * KPAD, rpb * KPAD)], dvm)
    zf16 = jnp.zeros((16,), jnp.float32)

    def zcol(i, _):
        colacc[pl.ds(i * 16, 16)] = zf16
        return 0

    jax.lax.fori_loop(0, n // 16, zcol, 0, unroll=8)

    def zrow(i, _):
        rowbuf[pl.ds(i * 16, 16)] = zf16
        return 0

    jax.lax.fori_loop(0, GB * n // 16, zrow, 0, unroll=8)

    def group(g, _):
        def scat(p, _):
            rr = p // (KPAD // 16)
            kk = p % (KPAD // 16)
            off = (g * GB + rr) * KPAD + kk * 16
            iv = ivm[pl.ds(off, 16)]
            dv = dvm[pl.ds(off, 16)]
            fl = iv + rr * n
            plsc.store_scatter(rowbuf, [fl], dv)
            plsc.addupdate_scatter(colacc, [iv], dv)
            return 0

        jax.lax.fori_loop(0, GB * (KPAD // 16), scat, 0)
        pltpu.sync_copy(rowbuf, a_hbm.at[pl.ds((base + g * GB) * n, GB * n)])

        def unscat(p, _):
            rr = p // (KPAD // 16)
            kk = p % (KPAD // 16)
            off = (g * GB + rr) * KPAD + kk * 16
            iv = ivm[pl.ds(off, 16)]
            fl = iv + rr * n
            plsc.store_scatter(rowbuf, [fl], zf16)
            return 0

        jax.lax.fori_loop(0, GB * (KPAD // 16), unscat, 0)
        return 0

    jax.lax.fori_loop(0, rpb // GB, group, 0)
    pltpu.sync_copy(colacc, cs_hbm.at[wid])


def _csum_body(p_ref, rs_ref, o_ref):
    s = 2.0 + jnp.sum(p_ref[...], axis=0, keepdims=True) + rs_ref[...]
    o_ref[...] = 1.0 / jnp.sqrt(s)


TB = 512


def _wn_body(a_ref, at_ref, dc_ref, dr_ref, o_ref):
    i = pl.program_id(0)
    j = pl.program_id(1)
    w = a_ref[...] + at_ref[...].T
    rid = i * TB + jax.lax.broadcasted_iota(jnp.int32, (TB, TB), 0)
    cid = j * TB + jax.lax.broadcasted_iota(jnp.int32, (TB, TB), 1)
    w = jnp.where(rid == cid, 2.0, w)
    o_ref[...] = (w * dc_ref[...]) * dr_ref[...]


def _cg_body(wn_ref, lab_ref, z_ref, lb_ref,
             x_s, r_s, p_s, g1_s, rs_s, *, n, c, bank):
    t = pl.program_id(0)
    j = pl.program_id(1)
    nblk = n // RB2

    @pl.when((t == 0) & (j == 0))
    def _():
        lab = lab_ref[...]
        ysum = jnp.sum(lab, axis=0, keepdims=True)
        y = lab / (ysum + 1e-8)
        x_s[...] = jnp.zeros((n, c), jnp.float32)
        r_s[...] = y
        p_s[...] = y
        rs_s[...] = jnp.sum(y * y, axis=0, keepdims=True)

    @pl.when(j < nblk)
    def _():
        wn = wn_ref[...]
        pv = p_s[...]
        g1 = jax.lax.dot_general(wn, pv, (((1,), (0,)), ((), ())),
                                 preferred_element_type=jnp.float32)
        g1_s[pl.ds(j * RB2, RB2), :] = g1

    @pl.when(j == nblk)
    def _():
        pv = p_s[...]
        ap = pv - ALPHA * g1_s[...]
        pap = jnp.sum(pv * ap, axis=0, keepdims=True)
        al = rs_s[...] / (pap + 1e-12)
        xn = x_s[...] + al * pv
        rn = r_s[...] - al * ap
        rsn = jnp.sum(rn * rn, axis=0, keepdims=True)
        beta = rsn / (rs_s[...] + 1e-12)
        pn = rn + beta * pv
        x_s[...] = xn
        r_s[...] = rn
        p_s[...] = pn
        rs_s[...] = rsn

        @pl.when(t == MAX_ITER - 1)
        def _():
            zb = xn[bank:, :]
            z_ref[...] = zb
            lb_ref[...] = jnp.argmax(zb, axis=1).astype(jnp.int32)[:, None]


def kernel(features, labels):
    n, d = features.shape
    c = labels.shape[1]
    bank = (3 * n) // 4
    nb1 = n // RB1
    nb2 = n // RB2

    xn = features / (jnp.linalg.norm(features, axis=1, keepdims=True) + 1e-12)

    ipad, dnpad, rowsum = pl.pallas_call(
        functools.partial(_topk_body, n=n),
        grid=(nb1,),
        in_specs=[
            pl.BlockSpec((RB1, d), lambda b: (b, 0)),
            pl.BlockSpec((n, d), lambda b: (0, 0)),
        ],
        out_specs=[
            pl.BlockSpec((RB1, KPAD), lambda b: (b, 0)),
            pl.BlockSpec((RB1, KPAD), lambda b: (b, 0)),
            pl.BlockSpec((RB1, 1), lambda b: (b, 0)),
        ],
        out_shape=[
            jax.ShapeDtypeStruct((n, KPAD), jnp.int32),
            jax.ShapeDtypeStruct((n, KPAD), jnp.float32),
            jax.ShapeDtypeStruct((n, 1), jnp.float32),
        ],
        scratch_shapes=[
            pltpu.VMEM((RB1, n), jnp.float32),
            pltpu.VMEM((KPAD, RB1), jnp.float32),
            pltpu.VMEM((KPAD, RB1), jnp.int32),
        ],
    )(xn, xn)

    rpb = n // NW
    a_flat, cspart = pl.kernel(
        functools.partial(_sc_build_body, n=n),
        out_type=[
            jax.ShapeDtypeStruct((n * n,), jnp.float32),
            jax.ShapeDtypeStruct((NW, n), jnp.float32),
        ],
        mesh=plsc.VectorSubcoreMesh(core_axis_name="c", subcore_axis_name="s"),
        compiler_params=pltpu.CompilerParams(needs_layout_passes=False),
        scratch_types=[
            pltpu.VMEM((rpb * KPAD,), jnp.int32),
            pltpu.VMEM((rpb * KPAD,), jnp.float32),
            pltpu.VMEM((GB * n,), jnp.float32),
            pltpu.VMEM((n,), jnp.float32),
        ],
    )(ipad.reshape(-1), dnpad.reshape(-1))
    a_mat = a_flat.reshape(n, n)

    dis_row = pl.pallas_call(
        _csum_body,
        out_shape=jax.ShapeDtypeStruct((1, n), jnp.float32),
    )(cspart, rowsum.reshape(1, n))

    dis_col = dis_row.reshape(n, 1)

    wn_mat = pl.pallas_call(
        _wn_body,
        grid=(n // TB, n // TB),
        in_specs=[
            pl.BlockSpec((TB, TB), lambda i, j: (i, j)),
            pl.BlockSpec((TB, TB), lambda i, j: (j, i)),
            pl.BlockSpec((TB, 1), lambda i, j: (i, 0)),
            pl.BlockSpec((1, TB), lambda i, j: (0, j)),
        ],
        out_specs=pl.BlockSpec((TB, TB), lambda i, j: (i, j)),
        out_shape=jax.ShapeDtypeStruct((n, n), jnp.float32),
    )(a_mat, a_mat, dis_col, dis_row)

    z, lb = pl.pallas_call(
        functools.partial(_cg_body, n=n, c=c, bank=bank),
        grid=(MAX_ITER, nb2 + 1),
        in_specs=[
            pl.BlockSpec((RB2, n), lambda t, j: (jnp.minimum(j, n // RB2 - 1), 0)),
            pl.BlockSpec((n, c), lambda t, j: (0, 0)),
        ],
        out_specs=[
            pl.BlockSpec((n - bank, c), lambda t, j: (0, 0)),
            pl.BlockSpec((n - bank, 1), lambda t, j: (0, 0)),
        ],
        out_shape=[
            jax.ShapeDtypeStruct((n - bank, c), jnp.float32),
            jax.ShapeDtypeStruct((n - bank, 1), jnp.int32),
        ],
        scratch_shapes=[
            pltpu.VMEM((n, c), jnp.float32),
            pltpu.VMEM((n, c), jnp.float32),
            pltpu.VMEM((n, c), jnp.float32),
            pltpu.VMEM((n, c), jnp.float32),
            pltpu.VMEM((1, c), jnp.float32),
        ],
    )(wn_mat, labels)

    return z, lb.reshape(n - bank)
